# baseline (speedup 1.0000x reference)
import functools

import jax
import jax.numpy as jnp
from jax import lax
from jax.experimental import pallas as pl
from jax.experimental.pallas import tpu as pltpu
from jax.experimental.pallas import tpu_sc as plsc

H = 224
W = 224
C = 96
B = 8
NPIX = H * W
NPLANES = B * C
NLANES = 16
NWORKERS = 32
WORKERS_PER_IMAGE = NWORKERS // B
CH_PER_WORKER = C // WORKERS_PER_IMAGE
CHUNKS_PER_ROW = W // NLANES
BLK_ROWS = 32
NBLK = H // BLK_ROWS
BLK_PX = BLK_ROWS * W
FMAX = NPIX - W - 2


def _splat(vec, lane):
    return jnp.full((NLANES,), vec[lane], dtype=jnp.float32)


def _bf16_round(v):
    u = plsc.bitcast(v, jnp.uint32)
    r = (u + jnp.uint32(0x7FFF) + ((u >> jnp.uint32(16)) & jnp.uint32(1))) \
        & jnp.uint32(0xFFFF0000)
    return plsc.bitcast(r, jnp.float32)


def _affine_body(x_hbm, t_hbm, out_hbm, tv, f_v, plane_v, ob0, ob1, sem0, sem1):
    cid = lax.axis_index("c")
    sid = lax.axis_index("s")
    wid = sid * 2 + cid
    b = wid // WORKERS_PER_IMAGE
    sub = wid % WORKERS_PER_IMAGE

    pltpu.sync_copy(t_hbm.at[b], tv)
    tvec = tv[...]

    i00 = _bf16_round(_splat(tvec, 0))
    i01 = _bf16_round(_splat(tvec, 1))
    i10 = _bf16_round(_splat(tvec, 2))
    i11 = _bf16_round(_splat(tvec, 3))
    ntx = _bf16_round(-_splat(tvec, 4))
    nty = _bf16_round(-_splat(tvec, 5))
    tpx = ntx * i00 + nty * i10
    tpy = ntx * i01 + nty * i11

    lim = jnp.float32(H - 2)
    zero = jnp.float32(0.0)
    cx0 = jnp.clip(tpx, zero, lim)
    cy0 = jnp.clip(tpy, zero, lim)
    dx0 = cx0 - cx0.astype(jnp.int32).astype(jnp.float32)
    dy0 = cy0 - cy0.astype(jnp.int32).astype(jnp.float32)
    w00 = (1.0 - dx0) * (1.0 - dy0)
    w10 = dx0 * (1.0 - dy0)
    w01 = (1.0 - dx0) * dy0
    w11 = dx0 * dy0

    lanes_f = lax.iota(jnp.int32, NLANES).astype(jnp.float32)

    def idx_row(y, carry):
        yf = jnp.full((NLANES,), y, dtype=jnp.int32).astype(jnp.float32)
        for j in range(CHUNKS_PER_ROW):
            xf = lanes_f + jnp.float32(j * NLANES)
            cx = jnp.clip((xf * i00 + yf * i10) + tpx, zero, lim)
            cy = jnp.clip((xf * i01 + yf * i11) + tpy, zero, lim)
            f = jnp.clip(cy.astype(jnp.int32) * W + cx.astype(jnp.int32),
                         0, FMAX)
            f_v[pl.ds(y * W + j * NLANES, NLANES)] = f
        return carry

    lax.fori_loop(0, H, idx_row, 0)

    out_bufs = (ob0, ob1)
    sems = (sem0, sem1)

    def chan_body(k, carry):
        plane = b * C + sub * CH_PER_WORKER + k
        pltpu.sync_copy(x_hbm.at[pl.ds(plane * NPIX, NPIX)], plane_v)

        copies = [None, None]
        for blk in range(NBLK):
            p = blk % 2
            buf = out_bufs[p]
            if copies[p] is not None:
                copies[p].wait()

            def blk_row(r, carry2):
                o = (blk * BLK_ROWS + r) * W
                for j in range(CHUNKS_PER_ROW):
                    f = f_v[pl.ds(o + j * NLANES, NLANES)]
                    g00 = plsc.load_gather(plane_v, [f])
                    g10 = plsc.load_gather(plane_v, [f + 1])
                    g01 = plsc.load_gather(plane_v, [f + W])
                    g11 = plsc.load_gather(plane_v, [f + (W + 1)])
                    val = w00 * g00 + w10 * g10 + w01 * g01 + w11 * g11
                    buf[pl.ds(r * W + j * NLANES, NLANES)] = val
                return carry2

            lax.fori_loop(0, BLK_ROWS, blk_row, 0)
            copies[p] = pltpu.async_copy(
                buf,
                out_hbm.at[pl.ds(plane * NPIX + blk * BLK_PX, BLK_PX)],
                sems[p])
        copies[0].wait()
        copies[1].wait()
        return carry

    lax.fori_loop(0, CH_PER_WORKER, chan_body, 0)


_affine_sc = functools.partial(
    pl.kernel,
    mesh=plsc.VectorSubcoreMesh(core_axis_name="c", subcore_axis_name="s"),
    out_type=jax.ShapeDtypeStruct((NPLANES * NPIX,), jnp.float32),
    compiler_params=pltpu.CompilerParams(needs_layout_passes=False),
    scratch_types=[
        pltpu.VMEM((NLANES,), jnp.float32),
        pltpu.VMEM((NPIX,), jnp.int32),
        pltpu.VMEM((NPIX,), jnp.float32),
        pltpu.VMEM((BLK_PX,), jnp.float32),
        pltpu.VMEM((BLK_PX,), jnp.float32),
        pltpu.SemaphoreType.DMA,
        pltpu.SemaphoreType.DMA,
    ],
)(_affine_body)


@jax.jit
def kernel(x, transform):
    x2 = x.reshape(NPLANES * NPIX)
    ainv = jnp.linalg.inv(transform[:, :4].reshape(B, 2, 2))
    params = jnp.concatenate([ainv.reshape(B, 4), transform[:, 4:6]], axis=1)
    params = jnp.pad(params, ((0, 0), (0, NLANES - 6)))
    out = _affine_sc(x2, params)
    return out.reshape(x.shape)


# speedup vs baseline: 1.7339x; 1.7339x over previous
import functools

import jax
import jax.numpy as jnp
from jax import lax
from jax.experimental import pallas as pl
from jax.experimental.pallas import tpu as pltpu
from jax.experimental.pallas import tpu_sc as plsc

H = 224
W = 224
C = 96
B = 8
NPIX = H * W
NPLANES = B * C
NLANES = 16
NWORKERS = 32
WORKERS_PER_IMAGE = NWORKERS // B
CH_PER_WORKER = C // WORKERS_PER_IMAGE
CHUNKS_PER_ROW = W // NLANES
BLK_ROWS = 32
NBLK = H // BLK_ROWS
BLK_PX = BLK_ROWS * W
FMAX = NPIX - W - 2
P_UNROLL = 4
PLANE_PAD = W + 2 * NLANES


def _splat(vec, lane):
    return jnp.full((NLANES,), vec[lane], dtype=jnp.float32)


def _bf16_round(v):
    u = plsc.bitcast(v, jnp.uint32)
    r = (u + jnp.uint32(0x7FFF) + ((u >> jnp.uint32(16)) & jnp.uint32(1))) \
        & jnp.uint32(0xFFFF0000)
    return plsc.bitcast(r, jnp.float32)


def _affine_body(x_hbm, t_hbm, out_hbm, tv, f_v, plane_v, ob0, ob1, sem0, sem1):
    cid = lax.axis_index("c")
    sid = lax.axis_index("s")
    wid = sid * 2 + cid
    b = wid // WORKERS_PER_IMAGE
    sub = wid % WORKERS_PER_IMAGE

    pltpu.sync_copy(t_hbm.at[b], tv)
    tvec = tv[...]

    i00 = _bf16_round(_splat(tvec, 0))
    i01 = _bf16_round(_splat(tvec, 1))
    i10 = _bf16_round(_splat(tvec, 2))
    i11 = _bf16_round(_splat(tvec, 3))
    ntx = _bf16_round(-_splat(tvec, 4))
    nty = _bf16_round(-_splat(tvec, 5))
    tpx = ntx * i00 + nty * i10
    tpy = ntx * i01 + nty * i11

    lim = jnp.float32(H - 2)
    zero = jnp.float32(0.0)
    cx0 = jnp.clip(tpx, zero, lim)
    cy0 = jnp.clip(tpy, zero, lim)
    dx0 = cx0 - cx0.astype(jnp.int32).astype(jnp.float32)
    dy0 = cy0 - cy0.astype(jnp.int32).astype(jnp.float32)
    w00 = (1.0 - dx0) * (1.0 - dy0)
    w10 = dx0 * (1.0 - dy0)
    w01 = (1.0 - dx0) * dy0
    w11 = dx0 * dy0

    lanes_f = lax.iota(jnp.int32, NLANES).astype(jnp.float32)

    def idx_row(y, carry):
        yf = jnp.full((NLANES,), y, dtype=jnp.int32).astype(jnp.float32)
        for j in range(CHUNKS_PER_ROW):
            xf = lanes_f + jnp.float32(j * NLANES)
            cx = jnp.clip((xf * i00 + yf * i10) + tpx, zero, lim)
            cy = jnp.clip((xf * i01 + yf * i11) + tpy, zero, lim)
            f = jnp.clip(cy.astype(jnp.int32) * W + cx.astype(jnp.int32),
                         0, FMAX)
            f_v[pl.ds(y * W + j * NLANES, NLANES)] = f
        return carry

    lax.fori_loop(0, H, idx_row, 0)

    out_bufs = (ob0, ob1)
    sems = (sem0, sem1)

    def chan_body(k, carry):
        plane = b * C + sub * CH_PER_WORKER + k
        pltpu.sync_copy(x_hbm.at[pl.ds(plane * NPIX, NPIX)],
                        plane_v.at[pl.ds(0, NPIX)])

        def p_body(m, carry2):
            for u in range(P_UNROLL):
                base = (m * P_UNROLL + u) * NLANES
                a = plane_v[pl.ds(base, NLANES)]
                bb = plane_v[pl.ds(base + 1, NLANES)]
                cc = plane_v[pl.ds(base + W, NLANES)]
                dd = plane_v[pl.ds(base + W + 1, NLANES)]
                plane_v[pl.ds(base, NLANES)] = (
                    w00 * a + w10 * bb + w01 * cc + w11 * dd)
            return carry2

        lax.fori_loop(0, NPIX // (NLANES * P_UNROLL), p_body, 0)

        copies = [None, None]
        for blk in range(NBLK):
            p = blk % 2
            buf = out_bufs[p]
            if copies[p] is not None:
                copies[p].wait()

            def blk_row(r, carry2):
                o = (blk * BLK_ROWS + r) * W
                for j in range(CHUNKS_PER_ROW):
                    f = f_v[pl.ds(o + j * NLANES, NLANES)]
                    val = plsc.load_gather(plane_v, [f])
                    buf[pl.ds(r * W + j * NLANES, NLANES)] = val
                return carry2

            lax.fori_loop(0, BLK_ROWS, blk_row, 0)
            copies[p] = pltpu.async_copy(
                buf,
                out_hbm.at[pl.ds(plane * NPIX + blk * BLK_PX, BLK_PX)],
                sems[p])
        copies[0].wait()
        copies[1].wait()
        return carry

    lax.fori_loop(0, CH_PER_WORKER, chan_body, 0)


_affine_sc = functools.partial(
    pl.kernel,
    mesh=plsc.VectorSubcoreMesh(core_axis_name="c", subcore_axis_name="s"),
    out_type=jax.ShapeDtypeStruct((NPLANES * NPIX,), jnp.float32),
    compiler_params=pltpu.CompilerParams(needs_layout_passes=False),
    scratch_types=[
        pltpu.VMEM((NLANES,), jnp.float32),
        pltpu.VMEM((NPIX,), jnp.int32),
        pltpu.VMEM((NPIX + PLANE_PAD,), jnp.float32),
        pltpu.VMEM((BLK_PX,), jnp.float32),
        pltpu.VMEM((BLK_PX,), jnp.float32),
        pltpu.SemaphoreType.DMA,
        pltpu.SemaphoreType.DMA,
    ],
)(_affine_body)


@jax.jit
def kernel(x, transform):
    x2 = x.reshape(NPLANES * NPIX)
    ainv = jnp.linalg.inv(transform[:, :4].reshape(B, 2, 2))
    params = jnp.concatenate([ainv.reshape(B, 4), transform[:, 4:6]], axis=1)
    params = jnp.pad(params, ((0, 0), (0, NLANES - 6)))
    out = _affine_sc(x2, params)
    return out.reshape(x.shape)


# speedup vs baseline: 2.2425x; 1.2934x over previous
import functools

import jax
import jax.numpy as jnp
from jax import lax
from jax.experimental import pallas as pl
from jax.experimental.pallas import tpu as pltpu
from jax.experimental.pallas import tpu_sc as plsc

H = 224
W = 224
C = 96
B = 8
NPIX = H * W
NPLANES = B * C
NLANES = 16
NWORKERS = 32
WORKERS_PER_IMAGE = NWORKERS // B
CH_PER_WORKER = C // WORKERS_PER_IMAGE
CHUNKS_PER_ROW = W // NLANES
BLK_ROWS = 32
NBLK = H // BLK_ROWS
BLK_PX = BLK_ROWS * W
FMAX = NPIX - W - 2
P_UNROLL = 4
PLANE_PAD = W + 2 * NLANES


def _splat(vec, lane):
    return jnp.full((NLANES,), vec[lane], dtype=jnp.float32)


def _bf16_round(v):
    u = plsc.bitcast(v, jnp.uint32)
    r = (u + jnp.uint32(0x7FFF) + ((u >> jnp.uint32(16)) & jnp.uint32(1))) \
        & jnp.uint32(0xFFFF0000)
    return plsc.bitcast(r, jnp.float32)


def _affine_body(x_hbm, t_hbm, out_hbm, tv, f_v, plane_v, ob0, ob1, sem0, sem1):
    cid = lax.axis_index("c")
    sid = lax.axis_index("s")
    wid = sid * 2 + cid
    b = wid // WORKERS_PER_IMAGE
    sub = wid % WORKERS_PER_IMAGE

    pltpu.sync_copy(t_hbm.at[b], tv)
    tvec = tv[...]

    i00 = _bf16_round(_splat(tvec, 0))
    i01 = _bf16_round(_splat(tvec, 1))
    i10 = _bf16_round(_splat(tvec, 2))
    i11 = _bf16_round(_splat(tvec, 3))
    ntx = _bf16_round(-_splat(tvec, 4))
    nty = _bf16_round(-_splat(tvec, 5))
    tpx = ntx * i00 + nty * i10
    tpy = ntx * i01 + nty * i11

    lim = jnp.float32(H - 2)
    zero = jnp.float32(0.0)
    cx0 = jnp.clip(tpx, zero, lim)
    cy0 = jnp.clip(tpy, zero, lim)
    dx0 = cx0 - cx0.astype(jnp.int32).astype(jnp.float32)
    dy0 = cy0 - cy0.astype(jnp.int32).astype(jnp.float32)
    w00 = (1.0 - dx0) * (1.0 - dy0)
    w10 = dx0 * (1.0 - dy0)
    w01 = (1.0 - dx0) * dy0
    w11 = dx0 * dy0

    lanes_f = lax.iota(jnp.int32, NLANES).astype(jnp.float32)

    def idx_row(y, carry):
        yf = jnp.full((NLANES,), y, dtype=jnp.int32).astype(jnp.float32)
        for j in range(CHUNKS_PER_ROW):
            xf = lanes_f + jnp.float32(j * NLANES)
            cx = jnp.clip((xf * i00 + yf * i10) + tpx, zero, lim)
            cy = jnp.clip((xf * i01 + yf * i11) + tpy, zero, lim)
            f = jnp.clip(cy.astype(jnp.int32) * W + cx.astype(jnp.int32),
                         0, FMAX)
            f_v[pl.ds(y * W + j * NLANES, NLANES)] = f
        return carry

    lax.fori_loop(0, H, idx_row, 0)

    out_bufs = (ob0, ob1)
    sems = (sem0, sem1)

    def chan_body(k, carry):
        plane = b * C + sub * CH_PER_WORKER + k
        pltpu.sync_copy(x_hbm.at[pl.ds(plane * NPIX, NPIX)],
                        plane_v.at[pl.ds(0, NPIX)])

        def p_body(m, carry2):
            bases = [(m * P_UNROLL + u) * NLANES for u in range(P_UNROLL)]
            a = [plane_v[pl.ds(o2, NLANES)] for o2 in bases]
            bb = [plane_v[pl.ds(o2 + 1, NLANES)] for o2 in bases]
            cc = [plane_v[pl.ds(o2 + W, NLANES)] for o2 in bases]
            dd = [plane_v[pl.ds(o2 + W + 1, NLANES)] for o2 in bases]
            vals = [w00 * a[u] + w10 * bb[u] + w01 * cc[u] + w11 * dd[u]
                    for u in range(P_UNROLL)]
            for u in range(P_UNROLL):
                plane_v[pl.ds(bases[u], NLANES)] = vals[u]
            return carry2

        lax.fori_loop(0, NPIX // (NLANES * P_UNROLL), p_body, 0)

        copies = [None, None]
        for blk in range(NBLK):
            p = blk % 2
            buf = out_bufs[p]
            if copies[p] is not None:
                copies[p].wait()

            def blk_row(r, carry2):
                o = (blk * BLK_ROWS + r) * W
                fs = [f_v[pl.ds(o + j * NLANES, NLANES)]
                      for j in range(CHUNKS_PER_ROW)]
                gs = [plsc.load_gather(plane_v, [fs[j]])
                      for j in range(CHUNKS_PER_ROW)]
                for j in range(CHUNKS_PER_ROW):
                    buf[pl.ds(r * W + j * NLANES, NLANES)] = gs[j]
                return carry2

            lax.fori_loop(0, BLK_ROWS, blk_row, 0)
            copies[p] = pltpu.async_copy(
                buf,
                out_hbm.at[pl.ds(plane * NPIX + blk * BLK_PX, BLK_PX)],
                sems[p])
        copies[0].wait()
        copies[1].wait()
        return carry

    lax.fori_loop(0, CH_PER_WORKER, chan_body, 0)


_affine_sc = functools.partial(
    pl.kernel,
    mesh=plsc.VectorSubcoreMesh(core_axis_name="c", subcore_axis_name="s"),
    out_type=jax.ShapeDtypeStruct((NPLANES * NPIX,), jnp.float32),
    compiler_params=pltpu.CompilerParams(needs_layout_passes=False),
    scratch_types=[
        pltpu.VMEM((NLANES,), jnp.float32),
        pltpu.VMEM((NPIX,), jnp.int32),
        pltpu.VMEM((NPIX + PLANE_PAD,), jnp.float32),
        pltpu.VMEM((BLK_PX,), jnp.float32),
        pltpu.VMEM((BLK_PX,), jnp.float32),
        pltpu.SemaphoreType.DMA,
        pltpu.SemaphoreType.DMA,
    ],
)(_affine_body)


@jax.jit
def kernel(x, transform):
    x2 = x.reshape(NPLANES * NPIX)
    ainv = jnp.linalg.inv(transform[:, :4].reshape(B, 2, 2))
    params = jnp.concatenate([ainv.reshape(B, 4), transform[:, 4:6]], axis=1)
    params = jnp.pad(params, ((0, 0), (0, NLANES - 6)))
    out = _affine_sc(x2, params)
    return out.reshape(x.shape)


# speedup vs baseline: 2.2442x; 1.0007x over previous
import functools

import jax
import jax.numpy as jnp
from jax import lax
from jax.experimental import pallas as pl
from jax.experimental.pallas import tpu as pltpu
from jax.experimental.pallas import tpu_sc as plsc

H = 224
W = 224
C = 96
B = 8
NPIX = H * W
NPLANES = B * C
NLANES = 16
NWORKERS = 32
WORKERS_PER_IMAGE = NWORKERS // B
CH_PER_WORKER = C // WORKERS_PER_IMAGE
CHUNKS_PER_ROW = W // NLANES
BLK_ROWS = 16
NBLK = H // BLK_ROWS
NBUF = 4
BLK_PX = BLK_ROWS * W
FMAX = NPIX - W - 2
P_UNROLL = 4
PLANE_PAD = W + 2 * NLANES


def _splat(vec, lane):
    return jnp.full((NLANES,), vec[lane], dtype=jnp.float32)


def _bf16_round(v):
    u = plsc.bitcast(v, jnp.uint32)
    r = (u + jnp.uint32(0x7FFF) + ((u >> jnp.uint32(16)) & jnp.uint32(1))) \
        & jnp.uint32(0xFFFF0000)
    return plsc.bitcast(r, jnp.float32)


def _affine_body(x_hbm, t_hbm, out_hbm, tv, f_v, plane_v,
                 ob0, ob1, ob2, ob3, sem0, sem1, sem2, sem3):
    cid = lax.axis_index("c")
    sid = lax.axis_index("s")
    wid = sid * 2 + cid
    b = wid // WORKERS_PER_IMAGE
    sub = wid % WORKERS_PER_IMAGE

    pltpu.sync_copy(t_hbm.at[b], tv)
    tvec = tv[...]

    i00 = _bf16_round(_splat(tvec, 0))
    i01 = _bf16_round(_splat(tvec, 1))
    i10 = _bf16_round(_splat(tvec, 2))
    i11 = _bf16_round(_splat(tvec, 3))
    ntx = _bf16_round(-_splat(tvec, 4))
    nty = _bf16_round(-_splat(tvec, 5))
    tpx = ntx * i00 + nty * i10
    tpy = ntx * i01 + nty * i11

    lim = jnp.float32(H - 2)
    zero = jnp.float32(0.0)
    cx0 = jnp.clip(tpx, zero, lim)
    cy0 = jnp.clip(tpy, zero, lim)
    dx0 = cx0 - cx0.astype(jnp.int32).astype(jnp.float32)
    dy0 = cy0 - cy0.astype(jnp.int32).astype(jnp.float32)
    w00 = (1.0 - dx0) * (1.0 - dy0)
    w10 = dx0 * (1.0 - dy0)
    w01 = (1.0 - dx0) * dy0
    w11 = dx0 * dy0

    lanes_f = lax.iota(jnp.int32, NLANES).astype(jnp.float32)

    def idx_row(y, carry):
        yf = jnp.full((NLANES,), y, dtype=jnp.int32).astype(jnp.float32)
        js = range(CHUNKS_PER_ROW)
        xfs = [lanes_f + jnp.float32(j * NLANES) for j in js]
        cxs = [jnp.clip((xfs[j] * i00 + yf * i10) + tpx, zero, lim) for j in js]
        cys = [jnp.clip((xfs[j] * i01 + yf * i11) + tpy, zero, lim) for j in js]
        fs = [jnp.clip(cys[j].astype(jnp.int32) * W + cxs[j].astype(jnp.int32),
                       0, FMAX) for j in js]
        for j in js:
            f_v[pl.ds(y * W + j * NLANES, NLANES)] = fs[j]
        return carry

    lax.fori_loop(0, H, idx_row, 0)

    out_bufs = (ob0, ob1, ob2, ob3)
    sems = (sem0, sem1, sem2, sem3)

    def chan_body(k, carry):
        plane = b * C + sub * CH_PER_WORKER + k
        pltpu.sync_copy(x_hbm.at[pl.ds(plane * NPIX, NPIX)],
                        plane_v.at[pl.ds(0, NPIX)])

        def p_body(m, carry2):
            bases = [(m * P_UNROLL + u) * NLANES for u in range(P_UNROLL)]
            a = [plane_v[pl.ds(o2, NLANES)] for o2 in bases]
            bb = [plane_v[pl.ds(o2 + 1, NLANES)] for o2 in bases]
            cc = [plane_v[pl.ds(o2 + W, NLANES)] for o2 in bases]
            dd = [plane_v[pl.ds(o2 + W + 1, NLANES)] for o2 in bases]
            vals = [w00 * a[u] + w10 * bb[u] + w01 * cc[u] + w11 * dd[u]
                    for u in range(P_UNROLL)]
            for u in range(P_UNROLL):
                plane_v[pl.ds(bases[u], NLANES)] = vals[u]
            return carry2

        lax.fori_loop(0, NPIX // (NLANES * P_UNROLL), p_body, 0)

        copies = [None] * NBUF
        for blk in range(NBLK):
            p = blk % NBUF
            buf = out_bufs[p]
            if copies[p] is not None:
                copies[p].wait()

            def blk_row(r, carry2):
                o = (blk * BLK_ROWS + r) * W
                fs = [f_v[pl.ds(o + j * NLANES, NLANES)]
                      for j in range(CHUNKS_PER_ROW)]
                gs = [plsc.load_gather(plane_v, [fs[j]])
                      for j in range(CHUNKS_PER_ROW)]
                for j in range(CHUNKS_PER_ROW):
                    buf[pl.ds(r * W + j * NLANES, NLANES)] = gs[j]
                return carry2

            lax.fori_loop(0, BLK_ROWS, blk_row, 0)
            copies[p] = pltpu.async_copy(
                buf,
                out_hbm.at[pl.ds(plane * NPIX + blk * BLK_PX, BLK_PX)],
                sems[p])
        for cp in copies:
            cp.wait()
        return carry

    lax.fori_loop(0, CH_PER_WORKER, chan_body, 0)


_affine_sc = functools.partial(
    pl.kernel,
    mesh=plsc.VectorSubcoreMesh(core_axis_name="c", subcore_axis_name="s"),
    out_type=jax.ShapeDtypeStruct((NPLANES * NPIX,), jnp.float32),
    compiler_params=pltpu.CompilerParams(needs_layout_passes=False),
    scratch_types=[
        pltpu.VMEM((NLANES,), jnp.float32),
        pltpu.VMEM((NPIX,), jnp.int32),
        pltpu.VMEM((NPIX + PLANE_PAD,), jnp.float32),
        pltpu.VMEM((BLK_PX,), jnp.float32),
        pltpu.VMEM((BLK_PX,), jnp.float32),
        pltpu.VMEM((BLK_PX,), jnp.float32),
        pltpu.VMEM((BLK_PX,), jnp.float32),
        pltpu.SemaphoreType.DMA,
        pltpu.SemaphoreType.DMA,
        pltpu.SemaphoreType.DMA,
        pltpu.SemaphoreType.DMA,
    ],
)(_affine_body)


@jax.jit
def kernel(x, transform):
    x2 = x.reshape(NPLANES * NPIX)
    ainv = jnp.linalg.inv(transform[:, :4].reshape(B, 2, 2))
    params = jnp.concatenate([ainv.reshape(B, 4), transform[:, 4:6]], axis=1)
    params = jnp.pad(params, ((0, 0), (0, NLANES - 6)))
    out = _affine_sc(x2, params)
    return out.reshape(x.shape)


# speedup vs baseline: 2.2626x; 1.0082x over previous
import functools

import jax
import jax.numpy as jnp
from jax import lax
from jax.experimental import pallas as pl
from jax.experimental.pallas import tpu as pltpu
from jax.experimental.pallas import tpu_sc as plsc

H = 224
W = 224
C = 96
B = 8
NPIX = H * W
NPLANES = B * C
NLANES = 16
NWORKERS = 32
WORKERS_PER_IMAGE = NWORKERS // B
CH_PER_WORKER = C // WORKERS_PER_IMAGE
CHUNKS_PER_ROW = W // NLANES
BLK_ROWS = 16
NBLK = H // BLK_ROWS
NBUF = 4
BLK_PX = BLK_ROWS * W
FMAX = NPIX - W - 2
P_UNROLL = 8
PLANE_PAD = W + 2 * NLANES


def _splat(vec, lane):
    return jnp.full((NLANES,), vec[lane], dtype=jnp.float32)


def _bf16_round(v):
    u = plsc.bitcast(v, jnp.uint32)
    r = (u + jnp.uint32(0x7FFF) + ((u >> jnp.uint32(16)) & jnp.uint32(1))) \
        & jnp.uint32(0xFFFF0000)
    return plsc.bitcast(r, jnp.float32)


def _affine_body(x_hbm, t_hbm, out_hbm, tv, f_v, plane_v,
                 ob0, ob1, ob2, ob3, sem0, sem1, sem2, sem3):
    cid = lax.axis_index("c")
    sid = lax.axis_index("s")
    wid = sid * 2 + cid
    b = wid // WORKERS_PER_IMAGE
    sub = wid % WORKERS_PER_IMAGE

    pltpu.sync_copy(t_hbm.at[b], tv)
    tvec = tv[...]

    i00 = _bf16_round(_splat(tvec, 0))
    i01 = _bf16_round(_splat(tvec, 1))
    i10 = _bf16_round(_splat(tvec, 2))
    i11 = _bf16_round(_splat(tvec, 3))
    ntx = _bf16_round(-_splat(tvec, 4))
    nty = _bf16_round(-_splat(tvec, 5))
    tpx = ntx * i00 + nty * i10
    tpy = ntx * i01 + nty * i11

    lim = jnp.float32(H - 2)
    zero = jnp.float32(0.0)
    cx0 = jnp.clip(tpx, zero, lim)
    cy0 = jnp.clip(tpy, zero, lim)
    dx0 = cx0 - cx0.astype(jnp.int32).astype(jnp.float32)
    dy0 = cy0 - cy0.astype(jnp.int32).astype(jnp.float32)
    w00 = (1.0 - dx0) * (1.0 - dy0)
    w10 = dx0 * (1.0 - dy0)
    w01 = (1.0 - dx0) * dy0
    w11 = dx0 * dy0

    lanes_f = lax.iota(jnp.int32, NLANES).astype(jnp.float32)

    def idx_row(y, carry):
        yf = jnp.full((NLANES,), y, dtype=jnp.int32).astype(jnp.float32)
        js = range(CHUNKS_PER_ROW)
        xfs = [lanes_f + jnp.float32(j * NLANES) for j in js]
        cxs = [jnp.clip((xfs[j] * i00 + yf * i10) + tpx, zero, lim) for j in js]
        cys = [jnp.clip((xfs[j] * i01 + yf * i11) + tpy, zero, lim) for j in js]
        fs = [jnp.clip(cys[j].astype(jnp.int32) * W + cxs[j].astype(jnp.int32),
                       0, FMAX) for j in js]
        for j in js:
            f_v[pl.ds(y * W + j * NLANES, NLANES)] = fs[j]
        return carry

    lax.fori_loop(0, H, idx_row, 0)

    out_bufs = (ob0, ob1, ob2, ob3)
    sems = (sem0, sem1, sem2, sem3)

    def chan_body(k, carry):
        plane = b * C + sub * CH_PER_WORKER + k
        pltpu.sync_copy(x_hbm.at[pl.ds(plane * NPIX, NPIX)],
                        plane_v.at[pl.ds(0, NPIX)])

        def p_body(m, carry2):
            bases = [(m * P_UNROLL + u) * NLANES for u in range(P_UNROLL)]
            a = [plane_v[pl.ds(o2, NLANES)] for o2 in bases]
            bb = [plane_v[pl.ds(o2 + 1, NLANES)] for o2 in bases]
            cc = [plane_v[pl.ds(o2 + W, NLANES)] for o2 in bases]
            dd = [plane_v[pl.ds(o2 + W + 1, NLANES)] for o2 in bases]
            vals = [w00 * a[u] + w10 * bb[u] + w01 * cc[u] + w11 * dd[u]
                    for u in range(P_UNROLL)]
            for u in range(P_UNROLL):
                plane_v[pl.ds(bases[u], NLANES)] = vals[u]
            return carry2

        lax.fori_loop(0, NPIX // (NLANES * P_UNROLL), p_body, 0)

        copies = [None] * NBUF
        for blk in range(NBLK):
            p = blk % NBUF
            buf = out_bufs[p]
            if copies[p] is not None:
                copies[p].wait()

            def blk_row(r, carry2):
                o = (blk * BLK_ROWS + r) * W
                fs = [f_v[pl.ds(o + j * NLANES, NLANES)]
                      for j in range(CHUNKS_PER_ROW)]
                gs = [plsc.load_gather(plane_v, [fs[j]])
                      for j in range(CHUNKS_PER_ROW)]
                for j in range(CHUNKS_PER_ROW):
                    buf[pl.ds(r * W + j * NLANES, NLANES)] = gs[j]
                return carry2

            lax.fori_loop(0, BLK_ROWS, blk_row, 0)
            copies[p] = pltpu.async_copy(
                buf,
                out_hbm.at[pl.ds(plane * NPIX + blk * BLK_PX, BLK_PX)],
                sems[p])
        for cp in copies:
            cp.wait()
        return carry

    lax.fori_loop(0, CH_PER_WORKER, chan_body, 0)


_affine_sc = functools.partial(
    pl.kernel,
    mesh=plsc.VectorSubcoreMesh(core_axis_name="c", subcore_axis_name="s"),
    out_type=jax.ShapeDtypeStruct((NPLANES * NPIX,), jnp.float32),
    compiler_params=pltpu.CompilerParams(needs_layout_passes=False),
    scratch_types=[
        pltpu.VMEM((NLANES,), jnp.float32),
        pltpu.VMEM((NPIX,), jnp.int32),
        pltpu.VMEM((NPIX + PLANE_PAD,), jnp.float32),
        pltpu.VMEM((BLK_PX,), jnp.float32),
        pltpu.VMEM((BLK_PX,), jnp.float32),
        pltpu.VMEM((BLK_PX,), jnp.float32),
        pltpu.VMEM((BLK_PX,), jnp.float32),
        pltpu.SemaphoreType.DMA,
        pltpu.SemaphoreType.DMA,
        pltpu.SemaphoreType.DMA,
        pltpu.SemaphoreType.DMA,
    ],
)(_affine_body)


@jax.jit
def kernel(x, transform):
    x2 = x.reshape(NPLANES * NPIX)
    ainv = jnp.linalg.inv(transform[:, :4].reshape(B, 2, 2))
    params = jnp.concatenate([ainv.reshape(B, 4), transform[:, 4:6]], axis=1)
    params = jnp.pad(params, ((0, 0), (0, NLANES - 6)))
    out = _affine_sc(x2, params)
    return out.reshape(x.shape)
